# Initial kernel scaffold; baseline (speedup 1.0000x reference)
import jax
import jax.numpy as jnp
from jax.experimental import pallas as pl


def kernel(pred_tensor, target_tensor):
    raise NotImplementedError("write your pallas kernel here")


# speedup vs baseline: 3.6951x; 3.6951x over previous
import jax
import jax.numpy as jnp
from jax import lax
from jax.experimental import pallas as pl
from jax.experimental.pallas import tpu as pltpu
from jax.experimental.pallas import tpu_sc as plsc
import functools

B, S, D = 256, 14, 14
C = 30
R = B * S * D
NC, NS, L = 2, 16, 16
NW = NC * NS
ROWS_PER_W = R // NW
WORDS_PER_W = ROWS_PER_W * C
GROUPS = ROWS_PER_W // L


def _sq(x):
    return x * x


def _sqrt16(x):
    i = lax.bitcast_convert_type(x, jnp.int32)
    i = jnp.int32(0x5F3759DF) - lax.shift_right_arithmetic(i, 1)
    y = lax.bitcast_convert_type(i, jnp.float32)
    for _ in range(4):
        y = y * (1.5 - 0.5 * x * y * y)
    return x * y


def _body(pred_hbm, targ_hbm, out_hbm, pred_v, targ_v, acc_v, sem_p, sem_t):
    wid = lax.axis_index("s") * NC + lax.axis_index("c")
    base = wid * WORDS_PER_W
    cp_p = pltpu.async_copy(pred_hbm.at[pl.ds(base, WORDS_PER_W)], pred_v, sem_p)
    cp_t = pltpu.async_copy(targ_hbm.at[pl.ds(base, WORDS_PER_W)], targ_v, sem_t)
    cp_p.wait()
    cp_t.wait()

    lane_off = lax.iota(jnp.int32, L) * C

    def group(g, acc):
        idx = lane_off + g * (L * C)
        p = [plsc.load_gather(pred_v, [idx + c]) for c in range(C)]
        t = [plsc.load_gather(targ_v, [idx + c]) for c in range(C)]

        conf = t[4]
        coo = jnp.where(conf > 0, 1.0, 0.0).astype(jnp.float32)
        noo = jnp.where(conf == 0, 1.0, 0.0).astype(jnp.float32)
        noo_row = _sq(p[4] - t[4]) + _sq(p[9] - t[9])

        t_xmin = t[0] - 0.5 * t[2]
        t_ymin = t[1] - 0.5 * t[3]
        t_xmax = t[0] + 0.5 * t[2]
        t_ymax = t[1] + 0.5 * t[3]
        area2 = t[2] * t[3]

        ious = []
        for k in (0, 5):
            xmin = p[k + 0] - 0.5 * p[k + 2]
            ymin = p[k + 1] - 0.5 * p[k + 3]
            xmax = p[k + 0] + 0.5 * p[k + 2]
            ymax = p[k + 1] + 0.5 * p[k + 3]
            ltx = jnp.maximum(xmin, t_xmin)
            lty = jnp.maximum(ymin, t_ymin)
            rbx = jnp.minimum(xmax, t_xmax)
            rby = jnp.minimum(ymax, t_ymax)
            whx = jnp.where(rbx - ltx < 0, 1.0, 0.0).astype(jnp.float32)
            why = jnp.where(rby - lty < 0, 1.0, 0.0).astype(jnp.float32)
            inter = whx * why
            area1 = p[k + 2] * p[k + 3]
            ious.append(inter / (area1 + area2 - inter))
        iou0, iou1 = ious
        isn0 = iou0 != iou0
        isn1 = iou1 != iou1
        sel = (iou1 > iou0) | (isn1 & (~isn0))

        rp = [jnp.where(sel, p[5 + j], p[j]) for j in range(5)]
        rt = [jnp.where(sel, t[5 + j], t[j]) for j in range(5)]

        contain = _sq(rp[4] - rt[4])
        loc = (_sq(rp[0] - rt[0]) + _sq(rp[1] - rt[1])
               + _sq(_sqrt16(rp[2]) - _sqrt16(rt[2]))
               + _sq(_sqrt16(rp[3]) - _sqrt16(rt[3])))
        cls = _sq(p[10] - t[10])
        for j in range(11, C):
            cls = cls + _sq(p[j] - t[j])

        return acc + (coo * (loc + 2.0 * contain + cls) + noo * noo_row)

    acc = lax.fori_loop(0, GROUPS, group, jnp.zeros((L,), jnp.float32))
    acc_v[...] = acc
    pltpu.sync_copy(acc_v, out_hbm.at[wid])


@jax.jit
def _sc_loss(pred_flat, targ_flat):
    mesh = plsc.VectorSubcoreMesh(core_axis_name="c", subcore_axis_name="s")
    fn = pl.kernel(
        _body,
        out_type=jax.ShapeDtypeStruct((NW, L), jnp.float32),
        mesh=mesh,
        scratch_types=[
            pltpu.VMEM((WORDS_PER_W,), jnp.float32),
            pltpu.VMEM((WORDS_PER_W,), jnp.float32),
            pltpu.VMEM((L,), jnp.float32),
            pltpu.SemaphoreType.DMA,
            pltpu.SemaphoreType.DMA,
        ],
        compiler_params=pltpu.CompilerParams(needs_layout_passes=False),
    )
    return fn(pred_flat, targ_flat)


def kernel(pred_tensor, target_tensor):
    n = pred_tensor.shape[0]
    partials = _sc_loss(pred_tensor.reshape(-1), target_tensor.reshape(-1))
    return jnp.sum(partials) / jnp.float32(n)


# speedup vs baseline: 9.5561x; 2.5862x over previous
import jax
import jax.numpy as jnp
from jax import lax
from jax.experimental import pallas as pl
from jax.experimental.pallas import tpu as pltpu
from jax.experimental.pallas import tpu_sc as plsc

NC, NS, L = 2, 16, 16
NW = NC * NS
GRID = 14
CELLS = GRID * GRID
C = 30
NB = 256
GROUPS = NB // L


def _sq(x):
    return x * x


def _sqrt16(x):
    i = lax.bitcast_convert_type(x, jnp.int32)
    i = jnp.int32(0x5F3759DF) - lax.shift_right_arithmetic(i, 1)
    y = lax.bitcast_convert_type(i, jnp.float32)
    for _ in range(4):
        y = y * (1.5 - 0.5 * x * y * y)
    return x * y


def _row_losses(p, t):
    conf = t[4]
    coo = jnp.where(conf > 0, 1.0, 0.0).astype(jnp.float32)
    noo = jnp.where(conf == 0, 1.0, 0.0).astype(jnp.float32)
    noo_row = _sq(p[4] - t[4]) + _sq(p[9] - t[9])

    t_xmin = t[0] - 0.5 * t[2]
    t_ymin = t[1] - 0.5 * t[3]
    t_xmax = t[0] + 0.5 * t[2]
    t_ymax = t[1] + 0.5 * t[3]
    area2 = t[2] * t[3]

    ious = []
    for k in (0, 5):
        xmin = p[k + 0] - 0.5 * p[k + 2]
        ymin = p[k + 1] - 0.5 * p[k + 3]
        xmax = p[k + 0] + 0.5 * p[k + 2]
        ymax = p[k + 1] + 0.5 * p[k + 3]
        ltx = jnp.maximum(xmin, t_xmin)
        lty = jnp.maximum(ymin, t_ymin)
        rbx = jnp.minimum(xmax, t_xmax)
        rby = jnp.minimum(ymax, t_ymax)
        whx = jnp.where(rbx - ltx < 0, 1.0, 0.0).astype(jnp.float32)
        why = jnp.where(rby - lty < 0, 1.0, 0.0).astype(jnp.float32)
        inter = whx * why
        area1 = p[k + 2] * p[k + 3]
        ious.append(inter / (area1 + area2 - inter))
    iou0, iou1 = ious
    isn0 = iou0 != iou0
    isn1 = iou1 != iou1
    sel = (iou1 > iou0) | (isn1 & (~isn0))

    rp = [jnp.where(sel, p[5 + j], p[j]) for j in range(5)]
    rt = [jnp.where(sel, t[5 + j], t[j]) for j in range(5)]

    contain = _sq(rp[4] - rt[4])
    loc = (_sq(rp[0] - rt[0]) + _sq(rp[1] - rt[1])
           + _sq(_sqrt16(rp[2]) - _sqrt16(rt[2]))
           + _sq(_sqrt16(rp[3]) - _sqrt16(rt[3])))
    cls = _sq(p[10] - t[10])
    for j in range(11, C):
        cls = cls + _sq(p[j] - t[j])

    return coo * (loc + 2.0 * contain + cls) + noo * noo_row


def _body(pred_hbm, targ_hbm, out_hbm, pv, tv, acc_v, sem_p, sem_t):
    w = lax.axis_index("s") * NC + lax.axis_index("c")
    start = jnp.where(w < 4, 7 * w, 6 * w + 4)
    count = jnp.where(w < 4, 7, 6)

    def cell_loop(k, acc):
        cell = start + k
        ci = cell // GRID
        cj = cell - ci * GRID
        cp_p = pltpu.async_copy(pred_hbm.at[ci, cj], pv, sem_p)
        cp_t = pltpu.async_copy(targ_hbm.at[ci, cj], tv, sem_t)
        cp_p.wait()
        cp_t.wait()

        def g_loop(g, a):
            b0 = g * L
            p = [pv[c, pl.ds(b0, L)] for c in range(C)]
            t = [tv[c, pl.ds(b0, L)] for c in range(C)]
            return a + _row_losses(p, t)

        return lax.fori_loop(0, GROUPS, g_loop, acc)

    acc = lax.fori_loop(0, count, cell_loop, jnp.zeros((L,), jnp.float32))
    acc_v[pl.ds(0, L)] = acc
    for k in range(1, 8):
        acc_v[pl.ds(k * L, L)] = jnp.zeros((L,), jnp.float32)
    pltpu.sync_copy(acc_v, out_hbm.at[w])


@jax.jit
def _sc_loss(pred_t, targ_t):
    mesh = plsc.VectorSubcoreMesh(core_axis_name="c", subcore_axis_name="s")
    fn = pl.kernel(
        _body,
        out_type=jax.ShapeDtypeStruct((NW, 128), jnp.float32),
        mesh=mesh,
        scratch_types=[
            pltpu.VMEM((C, NB), jnp.float32),
            pltpu.VMEM((C, NB), jnp.float32),
            pltpu.VMEM((128,), jnp.float32),
            pltpu.SemaphoreType.DMA,
            pltpu.SemaphoreType.DMA,
        ],
        compiler_params=pltpu.CompilerParams(
            use_tc_tiling_on_sc=True, needs_layout_passes=False),
    )
    return fn(pred_t, targ_t)


def kernel(pred_tensor, target_tensor):
    n = pred_tensor.shape[0]
    pt = jnp.transpose(pred_tensor, (1, 2, 3, 0))
    tt = jnp.transpose(target_tensor, (1, 2, 3, 0))
    partials = _sc_loss(pt, tt)
    return jnp.sum(partials) / jnp.float32(n)
